# baseline (speedup 1.0000x reference)
import jax
import jax.numpy as jnp
from jax import lax
from jax.experimental import pallas as pl
from jax.experimental.pallas import tpu as pltpu

_EPS = 1e-5
_VMEM_LIMIT = 50 * 1024 * 1024


def _stats_rows(y):
    return jnp.concatenate(
        [jnp.sum(y, axis=0, keepdims=True),
         jnp.sum(y * y, axis=0, keepdims=True)], axis=0)


def _fold_q(st, C, Q):
    tot = st[:, 0:C]
    for q in range(1, Q):
        tot = tot + st[:, q * C:(q + 1) * C]
    return tot


def _finalize(st_ref, g_ref, b_ref, count):
    tot = jnp.sum(st_ref[...], axis=0)
    mean = tot[0:1] / count
    var = jnp.maximum(tot[1:2] / count - mean * mean, 0.0)
    scale = g_ref[...] * lax.rsqrt(var + _EPS)
    shift = b_ref[...] - mean * scale
    return scale, shift


def _conv1_kernel(x_ref, w_ref, y_ref, st_ref):
    xb = x_ref[...].astype(jnp.bfloat16)
    y = jnp.dot(xb, w_ref[...].astype(jnp.bfloat16),
                preferred_element_type=jnp.float32)
    y_ref[...] = y.astype(jnp.bfloat16)
    st_ref[...] = _stats_rows(y)


def _make_conv2_kernel(W, HW, Q, count):
    def body(y1_ref, st_ref, g_ref, b_ref, cm_ref, w_ref, y2_ref, st2_ref):
        C1 = y1_ref.shape[-1]
        QC = Q * C1
        scale, shift = _finalize(st_ref, g_ref, b_ref, count)
        scale_q = jnp.concatenate([scale] * Q, axis=1)
        shift_q = jnp.concatenate([shift] * Q, axis=1)
        yq = jnp.concatenate([y1_ref[q] for q in range(Q)], axis=1)
        a = jnp.maximum(yq.astype(jnp.float32) * scale_q + shift_q, 0.0)
        cm = cm_ref[...]
        aL = a * cm[:, 1:2]
        aR = a * cm[:, 0:1]
        zrow = jnp.zeros((1, QC), jnp.float32)
        b_by_dj = (jnp.concatenate([zrow, aL[:HW - 1]], axis=0),
                   a,
                   jnp.concatenate([aR[1:], zrow], axis=0))
        slabs = []
        for di in range(3):
            for dj in range(3):
                r = (di - 1) * W
                src = b_by_dj[dj]
                if r == 0:
                    sh = src
                elif r > 0:
                    sh = jnp.concatenate(
                        [src[r:], jnp.zeros((r, QC), src.dtype)], axis=0)
                else:
                    sh = jnp.concatenate(
                        [jnp.zeros((-r, QC), src.dtype), src[:HW + r]],
                        axis=0)
                slabs.append(sh.astype(jnp.bfloat16))
        patch = jnp.concatenate(slabs, axis=1)
        y = jnp.dot(patch, w_ref[...],
                    preferred_element_type=jnp.float32)
        y2_ref[...] = y.astype(jnp.bfloat16)
        st2_ref[...] = _fold_q(_stats_rows(y), C1, Q)

    return body


def _make_conv3_stats_kernel(Q, C3, count):
    def body(y2_ref, st_ref, g_ref, b_ref, w_ref, a2_ref, st3_ref):
        C2 = st_ref.shape[-1]
        scale, shift = _finalize(st_ref, g_ref, b_ref, count)
        scale_q = jnp.concatenate([scale] * Q, axis=1)
        shift_q = jnp.concatenate([shift] * Q, axis=1)
        a = jnp.maximum(
            y2_ref[...].astype(jnp.float32) * scale_q + shift_q, 0.0)
        ab = a.astype(jnp.bfloat16)
        for h in range(Q // 2):
            a2_ref[h] = ab[:, h * 2 * C2:(h + 1) * 2 * C2]
        y = jnp.dot(ab, w_ref[...],
                    preferred_element_type=jnp.float32)
        st3_ref[...] = _fold_q(_stats_rows(y), C3, Q)

    return body


def _make_out_kernel(HW, count):
    def body(a2_ref, st_ref, g_ref, b_ref, w_ref, x_ref, o_ref):
        C3 = g_ref.shape[-1]
        scale, shift = _finalize(st_ref, g_ref, b_ref, count)
        y = jnp.dot(a2_ref[...], w_ref[...],
                    preferred_element_type=jnp.float32)
        o_ref[0:HW] = jnp.maximum(
            y[:, 0:C3] * scale + shift + x_ref[0:HW], 0.0)
        o_ref[HW:2 * HW] = jnp.maximum(
            y[:, C3:2 * C3] * scale + shift + x_ref[HW:2 * HW], 0.0)

    return body


def kernel(x, w1, w2, w3, g1, b1, g2, b2, g3, b3):
    N, Cin, H, W = x.shape
    C1 = w1.shape[1]
    C2 = w2.shape[2]
    C3 = w3.shape[1]
    HW = H * W
    M = N * HW
    Mf = float(M)
    Q = 4 if N % 4 == 0 else 2
    G = N // Q

    x2d = jnp.transpose(x, (0, 2, 3, 1)).reshape(M, Cin)
    j_idx = jnp.arange(HW, dtype=jnp.int32) % W
    col_mask = jnp.stack([j_idx > 0, j_idx < W - 1],
                         axis=1).astype(jnp.float32)
    eye_q = jnp.eye(Q, dtype=jnp.float32)
    eye_2 = jnp.eye(2, dtype=jnp.float32)
    w2q = (eye_q[None, :, None, :, None] *
           w2[:, None, :, None, :]).reshape(
               9 * Q * C1, Q * C2).astype(jnp.bfloat16)
    w3q = (eye_q[:, None, :, None] *
           w3[None, :, None, :]).reshape(Q * C2, Q * C3).astype(jnp.bfloat16)
    w3p = (eye_2[:, None, :, None] *
           w3[None, :, None, :]).reshape(2 * C2, 2 * C3).astype(jnp.bfloat16)

    cp = pltpu.CompilerParams(dimension_semantics=("parallel",),
                              vmem_limit_bytes=_VMEM_LIMIT)

    y1, st1 = pl.pallas_call(
        _conv1_kernel,
        out_shape=(jax.ShapeDtypeStruct((N, HW, C1), jnp.bfloat16),
                   jax.ShapeDtypeStruct((N, 2, C1), jnp.float32)),
        grid=(N,),
        in_specs=[pl.BlockSpec((HW, Cin), lambda n: (n, 0)),
                  pl.BlockSpec((Cin, C1), lambda n: (0, 0))],
        out_specs=(pl.BlockSpec((None, HW, C1), lambda n: (n, 0, 0)),
                   pl.BlockSpec((None, 2, C1), lambda n: (n, 0, 0))),
        compiler_params=cp,
        cost_estimate=pl.CostEstimate(
            flops=2 * M * Cin * C1, transcendentals=0,
            bytes_accessed=4 * M * Cin + 2 * M * C1),
    )(x2d, w1)

    y2, st2 = pl.pallas_call(
        _make_conv2_kernel(W, HW, Q, Mf),
        out_shape=(jax.ShapeDtypeStruct((G, HW, Q * C2), jnp.bfloat16),
                   jax.ShapeDtypeStruct((G, 2, C2), jnp.float32)),
        grid=(G,),
        in_specs=[pl.BlockSpec((Q, HW, C1), lambda g: (g, 0, 0)),
                  pl.BlockSpec((N, 2, C1), lambda g: (0, 0, 0)),
                  pl.BlockSpec((1, C1), lambda g: (0, 0)),
                  pl.BlockSpec((1, C1), lambda g: (0, 0)),
                  pl.BlockSpec((HW, 2), lambda g: (0, 0)),
                  pl.BlockSpec((9 * Q * C1, Q * C2), lambda g: (0, 0))],
        out_specs=(pl.BlockSpec((None, HW, Q * C2), lambda g: (g, 0, 0)),
                   pl.BlockSpec((None, 2, C2), lambda g: (g, 0, 0))),
        compiler_params=cp,
        cost_estimate=pl.CostEstimate(
            flops=2 * M * 9 * C1 * C2, transcendentals=0,
            bytes_accessed=2 * M * C1 + 2 * M * C2),
    )(y1, st1, g1, b1, col_mask, w2q)

    a2, st3 = pl.pallas_call(
        _make_conv3_stats_kernel(Q, C3, Mf),
        out_shape=(jax.ShapeDtypeStruct((N // 2, HW, 2 * C2), jnp.bfloat16),
                   jax.ShapeDtypeStruct((G, 2, C3), jnp.float32)),
        grid=(G,),
        in_specs=[pl.BlockSpec((None, HW, Q * C2), lambda g: (g, 0, 0)),
                  pl.BlockSpec((G, 2, C2), lambda g: (0, 0, 0)),
                  pl.BlockSpec((1, C2), lambda g: (0, 0)),
                  pl.BlockSpec((1, C2), lambda g: (0, 0)),
                  pl.BlockSpec((Q * C2, Q * C3), lambda g: (0, 0))],
        out_specs=(pl.BlockSpec((Q // 2, HW, 2 * C2), lambda g: (g, 0, 0)),
                   pl.BlockSpec((None, 2, C3), lambda g: (g, 0, 0))),
        compiler_params=cp,
        cost_estimate=pl.CostEstimate(
            flops=2 * M * C2 * C3, transcendentals=0,
            bytes_accessed=2 * M * C2 + 2 * M * C2),
    )(y2, st2, g2, b2, w3q)

    out2d = pl.pallas_call(
        _make_out_kernel(HW, Mf),
        out_shape=jax.ShapeDtypeStruct((M, C3), jnp.float32),
        grid=(N // 2,),
        in_specs=[pl.BlockSpec((None, HW, 2 * C2), lambda p: (p, 0, 0)),
                  pl.BlockSpec((G, 2, C3), lambda p: (0, 0, 0)),
                  pl.BlockSpec((1, C3), lambda p: (0, 0)),
                  pl.BlockSpec((1, C3), lambda p: (0, 0)),
                  pl.BlockSpec((2 * C2, 2 * C3), lambda p: (0, 0)),
                  pl.BlockSpec((2 * HW, C3), lambda p: (p, 0))],
        out_specs=pl.BlockSpec((2 * HW, C3), lambda p: (p, 0)),
        compiler_params=cp,
        cost_estimate=pl.CostEstimate(
            flops=2 * M * C2 * C3 + 3 * M * C3, transcendentals=0,
            bytes_accessed=2 * M * C2 + 8 * M * C3),
    )(a2, st3, g3, b3, w3p, x2d)

    return jnp.transpose(out2d.reshape(N, H, W, C3), (0, 3, 1, 2))


# speedup vs baseline: 1.1223x; 1.1223x over previous
import jax
import jax.numpy as jnp
from jax import lax
from jax.experimental import pallas as pl
from jax.experimental.pallas import tpu as pltpu

_EPS = 1e-5
_VMEM_LIMIT = 32 * 1024 * 1024


def _stats_rows(y):
    return jnp.concatenate(
        [jnp.sum(y, axis=0, keepdims=True),
         jnp.sum(y * y, axis=0, keepdims=True)], axis=0)


def _finalize(st_ref, g_ref, b_ref, count):
    tot = jnp.sum(st_ref[...], axis=0)
    mean = tot[0:1] / count
    var = jnp.maximum(tot[1:2] / count - mean * mean, 0.0)
    scale = g_ref[...] * lax.rsqrt(var + _EPS)
    shift = b_ref[...] - mean * scale
    return scale, shift


def _conv1_kernel(x_ref, w_ref, y_ref, st_ref):
    xb = x_ref[...].astype(jnp.bfloat16)
    y = jnp.dot(xb, w_ref[...].astype(jnp.bfloat16),
                preferred_element_type=jnp.float32)
    y_ref[...] = y.astype(jnp.bfloat16)
    st_ref[...] = _stats_rows(y)


def _make_conv2_kernel(W, HW, count):
    def body(y1_ref, st_ref, g_ref, b_ref, cm_ref, w_ref, y2_ref, st2_ref):
        C1 = y1_ref.shape[-1]
        scale, shift = _finalize(st_ref, g_ref, b_ref, count)
        a = jnp.maximum(y1_ref[...].astype(jnp.float32) * scale + shift, 0.0)
        cm = cm_ref[...]
        aL = a * cm[:, 1:2]
        aR = a * cm[:, 0:1]
        zrow = jnp.zeros((1, C1), jnp.float32)
        b_by_dj = (jnp.concatenate([zrow, aL[:HW - 1]], axis=0),
                   a,
                   jnp.concatenate([aR[1:], zrow], axis=0))
        slabs = []
        for di in range(3):
            for dj in range(3):
                r = (di - 1) * W
                src = b_by_dj[dj]
                if r == 0:
                    sh = src
                elif r > 0:
                    sh = jnp.concatenate(
                        [src[r:], jnp.zeros((r, C1), src.dtype)], axis=0)
                else:
                    sh = jnp.concatenate(
                        [jnp.zeros((-r, C1), src.dtype), src[:HW + r]],
                        axis=0)
                slabs.append(sh.astype(jnp.bfloat16))
        patch = jnp.concatenate(slabs, axis=1)
        y = jnp.dot(patch, w_ref[...].astype(jnp.bfloat16),
                    preferred_element_type=jnp.float32)
        y2_ref[...] = y.astype(jnp.bfloat16)
        st2_ref[...] = _stats_rows(y)

    return body


def _make_conv3_stats_kernel(count):
    def body(y2_ref, st_ref, g_ref, b_ref, w_ref, a2_ref, st3_ref):
        scale, shift = _finalize(st_ref, g_ref, b_ref, count)
        a = jnp.maximum(y2_ref[...].astype(jnp.float32) * scale + shift, 0.0)
        ab = a.astype(jnp.bfloat16)
        a2_ref[...] = ab
        y = jnp.dot(ab, w_ref[...].astype(jnp.bfloat16),
                    preferred_element_type=jnp.float32)
        st3_ref[...] = _stats_rows(y)

    return body


def _make_out_kernel(count):
    def body(a2_ref, st_ref, g_ref, b_ref, w_ref, x_ref, o_ref):
        scale, shift = _finalize(st_ref, g_ref, b_ref, count)
        y = jnp.dot(a2_ref[...], w_ref[...].astype(jnp.bfloat16),
                    preferred_element_type=jnp.float32)
        o_ref[...] = jnp.maximum(y * scale + shift + x_ref[...], 0.0)

    return body


def kernel(x, w1, w2, w3, g1, b1, g2, b2, g3, b3):
    N, Cin, H, W = x.shape
    C1 = w1.shape[1]
    C2 = w2.shape[2]
    C3 = w3.shape[1]
    HW = H * W
    M = N * HW
    Mf = float(M)

    x2d = jnp.transpose(x, (0, 2, 3, 1)).reshape(M, Cin)
    w2r = w2.reshape(9 * C1, C2)
    j_idx = jnp.arange(HW, dtype=jnp.int32) % W
    col_mask = jnp.stack([j_idx > 0, j_idx < W - 1],
                         axis=1).astype(jnp.float32)

    cp = pltpu.CompilerParams(dimension_semantics=("parallel",),
                              vmem_limit_bytes=_VMEM_LIMIT)

    y1, st1 = pl.pallas_call(
        _conv1_kernel,
        out_shape=(jax.ShapeDtypeStruct((N, HW, C1), jnp.bfloat16),
                   jax.ShapeDtypeStruct((N, 2, C1), jnp.float32)),
        grid=(N,),
        in_specs=[pl.BlockSpec((HW, Cin), lambda n: (n, 0)),
                  pl.BlockSpec((Cin, C1), lambda n: (0, 0))],
        out_specs=(pl.BlockSpec((None, HW, C1), lambda n: (n, 0, 0)),
                   pl.BlockSpec((None, 2, C1), lambda n: (n, 0, 0))),
        compiler_params=cp,
        cost_estimate=pl.CostEstimate(
            flops=2 * M * Cin * C1, transcendentals=0,
            bytes_accessed=4 * M * Cin + 2 * M * C1),
    )(x2d, w1)

    y2, st2 = pl.pallas_call(
        _make_conv2_kernel(W, HW, Mf),
        out_shape=(jax.ShapeDtypeStruct((N, HW, C2), jnp.bfloat16),
                   jax.ShapeDtypeStruct((N, 2, C2), jnp.float32)),
        grid=(N,),
        in_specs=[pl.BlockSpec((None, HW, C1), lambda n: (n, 0, 0)),
                  pl.BlockSpec((N, 2, C1), lambda n: (0, 0, 0)),
                  pl.BlockSpec((1, C1), lambda n: (0, 0)),
                  pl.BlockSpec((1, C1), lambda n: (0, 0)),
                  pl.BlockSpec((HW, 2), lambda n: (0, 0)),
                  pl.BlockSpec((9 * C1, C2), lambda n: (0, 0))],
        out_specs=(pl.BlockSpec((None, HW, C2), lambda n: (n, 0, 0)),
                   pl.BlockSpec((None, 2, C2), lambda n: (n, 0, 0))),
        compiler_params=cp,
        cost_estimate=pl.CostEstimate(
            flops=2 * M * 9 * C1 * C2, transcendentals=0,
            bytes_accessed=2 * M * C1 + 2 * M * C2),
    )(y1, st1, g1, b1, col_mask, w2r)

    a2, st3 = pl.pallas_call(
        _make_conv3_stats_kernel(Mf),
        out_shape=(jax.ShapeDtypeStruct((N, HW, C2), jnp.bfloat16),
                   jax.ShapeDtypeStruct((N, 2, C3), jnp.float32)),
        grid=(N,),
        in_specs=[pl.BlockSpec((None, HW, C2), lambda n: (n, 0, 0)),
                  pl.BlockSpec((N, 2, C2), lambda n: (0, 0, 0)),
                  pl.BlockSpec((1, C2), lambda n: (0, 0)),
                  pl.BlockSpec((1, C2), lambda n: (0, 0)),
                  pl.BlockSpec((C2, C3), lambda n: (0, 0))],
        out_specs=(pl.BlockSpec((None, HW, C2), lambda n: (n, 0, 0)),
                   pl.BlockSpec((None, 2, C3), lambda n: (n, 0, 0))),
        compiler_params=cp,
        cost_estimate=pl.CostEstimate(
            flops=2 * M * C2 * C3, transcendentals=0,
            bytes_accessed=2 * M * C2 + 2 * M * C2),
    )(y2, st2, g2, b2, w3)

    out2d = pl.pallas_call(
        _make_out_kernel(Mf),
        out_shape=jax.ShapeDtypeStruct((M, C3), jnp.float32),
        grid=(N,),
        in_specs=[pl.BlockSpec((None, HW, C2), lambda n: (n, 0, 0)),
                  pl.BlockSpec((N, 2, C3), lambda n: (0, 0, 0)),
                  pl.BlockSpec((1, C3), lambda n: (0, 0)),
                  pl.BlockSpec((1, C3), lambda n: (0, 0)),
                  pl.BlockSpec((C2, C3), lambda n: (0, 0)),
                  pl.BlockSpec((HW, C3), lambda n: (n, 0))],
        out_specs=pl.BlockSpec((HW, C3), lambda n: (n, 0)),
        compiler_params=cp,
        cost_estimate=pl.CostEstimate(
            flops=2 * M * C2 * C3 + 3 * M * C3, transcendentals=0,
            bytes_accessed=2 * M * C2 + 8 * M * C3),
    )(a2, st3, g3, b3, w3, x2d)

    return jnp.transpose(out2d.reshape(N, H, W, C3), (0, 3, 1, 2))
